# Initial kernel scaffold; baseline (speedup 1.0000x reference)
import jax
import jax.numpy as jnp
from jax.experimental import pallas as pl


def kernel(aa_indices, mod_x, W_mod, aa_table):
    raise NotImplementedError("write your pallas kernel here")


# speedup vs baseline: 4.3454x; 4.3454x over previous
import jax
import jax.numpy as jnp
from jax.experimental import pallas as pl

B, L = 4096, 64
MOD_IN = 109
K = 6
MOD_OUT = 8
OUT_FEATURES = 256
AA_DIM = OUT_FEATURES - MOD_OUT
VOCAB = 128

R = 1024
N = B * L


def _body(idx_ref, mod_ref, wa_ref, wb_ref, out_ref):
    idx = idx_ref[0, 0, :]
    iota = jax.lax.broadcasted_iota(jnp.int32, (R, VOCAB), 1)
    one_hot = (idx[:, None] == iota).astype(jnp.bfloat16)
    mod = mod_ref[...].astype(jnp.bfloat16)
    acc = jnp.dot(one_hot, wa_ref[...], preferred_element_type=jnp.float32)
    acc += jnp.dot(mod, wb_ref[...], preferred_element_type=jnp.float32)
    out_ref[...] = acc


def kernel(aa_indices, mod_x, W_mod, aa_table):
    idx = aa_indices.reshape(N // R, 1, R).astype(jnp.int32)
    mod = mod_x.reshape(N, MOD_IN)

    wa = jnp.zeros((VOCAB, OUT_FEATURES), jnp.float32).at[:, :AA_DIM].set(aa_table)
    wb = jnp.zeros((MOD_IN, OUT_FEATURES), jnp.float32)
    wb = wb.at[jnp.arange(K), AA_DIM + jnp.arange(K)].set(1.0)
    wb = wb.at[K:, AA_DIM + K:].set(W_mod)
    wa = wa.astype(jnp.bfloat16)
    wb = wb.astype(jnp.bfloat16)

    out = pl.pallas_call(
        _body,
        grid=(N // R,),
        in_specs=[
            pl.BlockSpec((1, 1, R), lambda i: (i, 0, 0)),
            pl.BlockSpec((R, MOD_IN), lambda i: (i, 0)),
            pl.BlockSpec((VOCAB, OUT_FEATURES), lambda i: (0, 0)),
            pl.BlockSpec((MOD_IN, OUT_FEATURES), lambda i: (0, 0)),
        ],
        out_specs=pl.BlockSpec((R, OUT_FEATURES), lambda i: (i, 0)),
        out_shape=jax.ShapeDtypeStruct((N, OUT_FEATURES), jnp.float32),
    )(idx, mod, wa, wb)
    return out.reshape(B, L, OUT_FEATURES)
